# baseline (speedup 1.0000x reference)
import functools

import jax
import jax.numpy as jnp
from jax import lax
from jax.experimental import pallas as pl
from jax.experimental.pallas import tpu as pltpu
from jax.experimental.pallas import tpu_sc as plsc

_BATCH = 16384
_VOCAB = 1000000
_DIM = 8
_TW = 128
_NC = 2
_NS = 16
_L = 16
_NW = _NC * _NS
_BPW = _BATCH // _NW
_G = _BPW // _L
_W = 16

_mesh = plsc.VectorSubcoreMesh(
    core_axis_name="c", subcore_axis_name="s", num_cores=_NC, num_subcores=_NS
)


@functools.partial(
    pl.kernel,
    out_type=jax.ShapeDtypeStruct((_BATCH,), jnp.float32),
    mesh=_mesh,
    compiler_params=pltpu.CompilerParams(needs_layout_passes=False),
    scratch_types=[
        pltpu.VMEM((_BPW,), jnp.int32),
        pltpu.VMEM((_BPW,), jnp.int32),
        pltpu.VMEM((2 * _L, _DIM, _TW), jnp.float32),
        pltpu.VMEM((2 * _L, _DIM, _TW), jnp.float32),
        pltpu.VMEM((_BPW,), jnp.float32),
        pltpu.SemaphoreType.DMA,
        pltpu.SemaphoreType.DMA,
        pltpu.SemaphoreType.DMA,
        pltpu.SemaphoreType.DMA,
    ],
)
def _mf_dot(iu_hbm, iv_hbm, ut_hbm, vt_hbm, out_hbm,
            iu_v, iv_v, u_t, v_t, o_v, sem_u0, sem_v0, sem_u1, sem_v1):
    wid = lax.axis_index("s") * _NC + lax.axis_index("c")
    base = wid * _BPW

    pltpu.sync_copy(iu_hbm.at[pl.ds(base, _BPW)], iu_v)
    pltpu.sync_copy(iv_hbm.at[pl.ds(base, _BPW)], iv_v)

    lanes = lax.iota(jnp.int32, _L)
    c127 = jnp.full((_L,), _TW - 1, jnp.int32)

    def fire(g, slot, sem_u, sem_v):
        uvec = iu_v[pl.ds(g * _L, _L)]
        vvec = iv_v[pl.ds(g * _L, _L)]
        for l in range(_L):
            cu = pl.multiple_of(
                lax.shift_left(lax.shift_right_logical(uvec[l], 7), 7), _TW)
            cv = pl.multiple_of(
                lax.shift_left(lax.shift_right_logical(vvec[l], 7), 7), _TW)
            pltpu.async_copy(
                ut_hbm.at[:, pl.ds(cu, _TW)], u_t.at[slot + l], sem_u)
            pltpu.async_copy(
                vt_hbm.at[:, pl.ds(cv, _TW)], v_t.at[slot + l], sem_v)

    def drain(sem_u, sem_v):
        for l in range(_L):
            pltpu.make_async_copy(
                ut_hbm.at[:, pl.ds(0, _TW)], u_t.at[l], sem_u).wait()
            pltpu.make_async_copy(
                vt_hbm.at[:, pl.ds(0, _TW)], v_t.at[l], sem_v).wait()

    def compute(g, slot):
        uvec = iu_v[pl.ds(g * _L, _L)]
        vvec = iv_v[pl.ds(g * _L, _L)]
        ucol = lax.bitwise_and(uvec, c127)
        vcol = lax.bitwise_and(vvec, c127)
        tid = slot + lanes
        acc = jnp.zeros((_L,), jnp.float32)
        for d in range(_DIM):
            dd = jnp.full((_L,), d, jnp.int32)
            ud = plsc.load_gather(u_t, [tid, dd, ucol])
            vd = plsc.load_gather(v_t, [tid, dd, vcol])
            acc = acc + ud * vd
        o_v[pl.ds(g * _L, _L)] = acc

    fire(0, 0, sem_u0, sem_v0)

    def body(h, carry):
        g0 = h * 2
        fire(g0 + 1, _L, sem_u1, sem_v1)
        drain(sem_u0, sem_v0)
        compute(g0, 0)

        @pl.when(g0 + 2 < _G)
        def _():
            fire(g0 + 2, 0, sem_u0, sem_v0)

        drain(sem_u1, sem_v1)
        compute(g0 + 1, _L)
        return carry

    lax.fori_loop(0, _G // 2, body, 0, unroll=False)

    pltpu.sync_copy(o_v, out_hbm.at[pl.ds(base, _BPW)])


def kernel(x, U, V):
    xt = x.T
    return _mf_dot(xt[0], xt[1], U.T, V.T)


# speedup vs baseline: 1.0005x; 1.0005x over previous
import functools

import jax
import jax.numpy as jnp
from jax import lax
from jax.experimental import pallas as pl
from jax.experimental.pallas import tpu as pltpu
from jax.experimental.pallas import tpu_sc as plsc

_BATCH = 16384
_DIM = 8
_TW = 128
_NC = 2
_NS = 16
_L = 16
_NW = _NC * _NS
_BPW = _BATCH // _NW
_G = _BPW // _L

_mesh = plsc.VectorSubcoreMesh(
    core_axis_name="c", subcore_axis_name="s", num_cores=_NC, num_subcores=_NS
)


@functools.partial(
    pl.kernel,
    out_type=jax.ShapeDtypeStruct((_BATCH,), jnp.float32),
    mesh=_mesh,
    compiler_params=pltpu.CompilerParams(needs_layout_passes=False),
    scratch_types=[
        pltpu.VMEM((_BPW,), jnp.int32),
        pltpu.VMEM((_BPW,), jnp.int32),
        pltpu.VMEM((2 * _L, _DIM, _TW), jnp.float32),
        pltpu.VMEM((2 * _L, _DIM, _TW), jnp.float32),
        pltpu.VMEM((_BPW,), jnp.float32),
        pltpu.SemaphoreType.DMA,
        pltpu.SemaphoreType.DMA,
        pltpu.SemaphoreType.DMA,
        pltpu.SemaphoreType.DMA,
    ],
)
def _mf_dot(iu_hbm, iv_hbm, ut_hbm, vt_hbm, out_hbm,
            iu_v, iv_v, u_t, v_t, o_v, sem_u0, sem_v0, sem_u1, sem_v1):
    wid = lax.axis_index("s") * _NC + lax.axis_index("c")
    base = wid * _BPW

    pltpu.sync_copy(iu_hbm.at[pl.ds(base, _BPW)], iu_v)
    pltpu.sync_copy(iv_hbm.at[pl.ds(base, _BPW)], iv_v)

    lanes = lax.iota(jnp.int32, _L)
    c127 = jnp.full((_L,), _TW - 1, jnp.int32)

    def fire(g, slot, sem_u, sem_v):
        uvec = iu_v[pl.ds(g * _L, _L)]
        vvec = iv_v[pl.ds(g * _L, _L)]
        for l in range(_L):
            cu = pl.multiple_of(
                lax.shift_left(lax.shift_right_logical(uvec[l], 7), 7), _TW)
            cv = pl.multiple_of(
                lax.shift_left(lax.shift_right_logical(vvec[l], 7), 7), _TW)
            pltpu.async_copy(
                ut_hbm.at[:, pl.ds(cu, _TW)], u_t.at[slot + l], sem_u)
            pltpu.async_copy(
                vt_hbm.at[:, pl.ds(cv, _TW)], v_t.at[slot + l], sem_v)

    def drain(sem_u, sem_v):
        for l in range(_L):
            pltpu.make_async_copy(
                ut_hbm.at[:, pl.ds(0, _TW)], u_t.at[l], sem_u).wait()
            pltpu.make_async_copy(
                vt_hbm.at[:, pl.ds(0, _TW)], v_t.at[l], sem_v).wait()

    def compute(g, slot):
        uvec = iu_v[pl.ds(g * _L, _L)]
        vvec = iv_v[pl.ds(g * _L, _L)]
        ucol = lax.bitwise_and(uvec, c127)
        vcol = lax.bitwise_and(vvec, c127)
        tid = slot + lanes
        acc = jnp.zeros((_L,), jnp.float32)
        for d in range(_DIM):
            dd = jnp.full((_L,), d, jnp.int32)
            ud = plsc.load_gather(u_t, [tid, dd, ucol])
            vd = plsc.load_gather(v_t, [tid, dd, vcol])
            acc = acc + ud * vd
        o_v[pl.ds(g * _L, _L)] = acc

    fire(0, 0, sem_u0, sem_v0)

    def body(h, carry):
        g0 = h * 2
        fire(g0 + 1, _L, sem_u1, sem_v1)
        drain(sem_u0, sem_v0)
        compute(g0, 0)

        @pl.when(g0 + 2 < _G)
        def _():
            fire(g0 + 2, 0, sem_u0, sem_v0)

        drain(sem_u1, sem_v1)
        compute(g0 + 1, _L)
        return carry

    lax.fori_loop(0, _G // 2, body, 0, unroll=False)

    pltpu.sync_copy(o_v, out_hbm.at[pl.ds(base, _BPW)])


def kernel(x, U, V):
    xt = x.T
    return _mf_dot(xt[0], xt[1], U.T, V.T)


# speedup vs baseline: 1.0283x; 1.0278x over previous
import functools

import jax
import jax.numpy as jnp
from jax import lax
from jax.experimental import pallas as pl
from jax.experimental.pallas import tpu as pltpu
from jax.experimental.pallas import tpu_sc as plsc

_BATCH = 16384
_DIM = 8
_TW = 128
_NC = 2
_NS = 16
_L = 16
_NW = _NC * _NS
_BPW = _BATCH // _NW
_G = _BPW // _L

_mesh = plsc.VectorSubcoreMesh(
    core_axis_name="c", subcore_axis_name="s", num_cores=_NC, num_subcores=_NS
)


@functools.partial(
    pl.kernel,
    out_type=jax.ShapeDtypeStruct((_BATCH,), jnp.float32),
    mesh=_mesh,
    compiler_params=pltpu.CompilerParams(needs_layout_passes=False),
    scratch_types=[
        pltpu.VMEM((2, _BPW), jnp.int32),
        pltpu.VMEM((2 * _L, _DIM, _TW), jnp.float32),
        pltpu.VMEM((2 * _L, _DIM, _TW), jnp.float32),
        pltpu.VMEM((_BPW,), jnp.float32),
        pltpu.SemaphoreType.DMA,
        pltpu.SemaphoreType.DMA,
        pltpu.SemaphoreType.DMA,
        pltpu.SemaphoreType.DMA,
    ],
)
def _mf_dot(xt_hbm, ut_hbm, vt_hbm, out_hbm,
            x_v, u_t, v_t, o_v, sem_u0, sem_v0, sem_u1, sem_v1):
    wid = lax.axis_index("s") * _NC + lax.axis_index("c")
    base = pl.multiple_of(wid * _BPW, _TW)

    pltpu.sync_copy(xt_hbm.at[:, pl.ds(base, _BPW)], x_v)

    lanes = lax.iota(jnp.int32, _L)
    c127 = jnp.full((_L,), _TW - 1, jnp.int32)

    def fire(g, slot, sem_u, sem_v):
        uvec = x_v[0, pl.ds(g * _L, _L)]
        vvec = x_v[1, pl.ds(g * _L, _L)]
        for l in range(_L):
            cu = pl.multiple_of(
                lax.shift_left(lax.shift_right_logical(uvec[l], 7), 7), _TW)
            cv = pl.multiple_of(
                lax.shift_left(lax.shift_right_logical(vvec[l], 7), 7), _TW)
            pltpu.async_copy(
                ut_hbm.at[:, pl.ds(cu, _TW)], u_t.at[slot + l], sem_u)
            pltpu.async_copy(
                vt_hbm.at[:, pl.ds(cv, _TW)], v_t.at[slot + l], sem_v)

    def drain(sem_u, sem_v):
        for l in range(_L):
            pltpu.make_async_copy(
                ut_hbm.at[:, pl.ds(0, _TW)], u_t.at[l], sem_u).wait()
            pltpu.make_async_copy(
                vt_hbm.at[:, pl.ds(0, _TW)], v_t.at[l], sem_v).wait()

    def compute(g, slot):
        uvec = x_v[0, pl.ds(g * _L, _L)]
        vvec = x_v[1, pl.ds(g * _L, _L)]
        ucol = lax.bitwise_and(uvec, c127)
        vcol = lax.bitwise_and(vvec, c127)
        tid = slot + lanes
        acc = jnp.zeros((_L,), jnp.float32)
        for d in range(_DIM):
            dd = jnp.full((_L,), d, jnp.int32)
            ud = plsc.load_gather(u_t, [tid, dd, ucol])
            vd = plsc.load_gather(v_t, [tid, dd, vcol])
            acc = acc + ud * vd
        o_v[pl.ds(g * _L, _L)] = acc

    fire(0, 0, sem_u0, sem_v0)

    def body(h, carry):
        g0 = h * 2
        fire(g0 + 1, _L, sem_u1, sem_v1)
        drain(sem_u0, sem_v0)
        compute(g0, 0)

        @pl.when(g0 + 2 < _G)
        def _():
            fire(g0 + 2, 0, sem_u0, sem_v0)

        drain(sem_u1, sem_v1)
        compute(g0 + 1, _L)
        return carry

    lax.fori_loop(0, _G // 2, body, 0, unroll=False)

    pltpu.sync_copy(o_v, out_hbm.at[pl.ds(base, _BPW)])


def kernel(x, U, V):
    return _mf_dot(x.T, U.T, V.T)
